# baseline (speedup 1.0000x reference)
import functools

import jax
import jax.numpy as jnp
from jax import lax
from jax.experimental import pallas as pl
from jax.experimental.pallas import tpu as pltpu
from jax.experimental.pallas import tpu_sc as plsc

KNN = 16
_BIG = 3.0e38

_NC, _NS = 2, 16
_NW = _NC * _NS


def _rn_bf16(v):
    u = lax.bitcast_convert_type(v, jnp.int32)
    odd = lax.shift_right_logical(u, 16) & 1
    r = (u + 32767 + odd) & jnp.int32(-65536)
    return lax.bitcast_convert_type(r, jnp.float32)


def _topk_body(xsr_ref, ysr_ref, xsc_ref, ysc_ref, idx_ref, *, n, rows):
    b = pl.program_id(0)
    grps = n // 128
    xr = xsr_ref[0].reshape(rows, 1, 1)
    yr = ysr_ref[0].reshape(rows, 1, 1)
    xc = xsc_ref[0].reshape(1, grps, 128)
    yc = ysc_ref[0].reshape(1, grps, 128)
    sqr = xr * xr + yr * yr
    sqc = xc * xc + yc * yc
    xrb, yrb, xcb, ycb = _rn_bf16(xr), _rn_bf16(yr), _rn_bf16(xc), _rn_bf16(yc)
    d2 = sqr + sqc - 2.0 * (xrb * xcb + yrb * ycb)

    a_iota = lax.broadcasted_iota(jnp.int32, (rows, grps, 128), 1)
    l_iota = lax.broadcasted_iota(jnp.int32, (rows, grps, 128), 2)
    col3 = a_iota * 128 + l_iota
    lane_k = lax.broadcasted_iota(jnp.int32, (rows, KNN), 1)
    _SENT = jnp.int32(1 << 30)

    cand_v, cand_j = [], []
    gm = jnp.full((rows, 1, 128), -_BIG, jnp.float32)
    for _ in range(4):
        dm = jnp.where(d2 > gm, d2, _BIG)
        m = jnp.min(dm, axis=1, keepdims=True)
        asel = jnp.where(dm == m, a_iota, grps)
        a_star = jnp.min(asel, axis=1, keepdims=True)
        cand_v.append(m)
        cand_j.append(a_star * 128 + l_iota[:, :1, :])
        gm = m
    C = jnp.concatenate(cand_v, axis=1)
    J = jnp.concatenate(cand_j, axis=1)

    def bbody(k, carry):
        cw, acc_i, acc_v = carry
        m = jnp.min(cw, axis=(1, 2), keepdims=True)
        jm = jnp.min(jnp.where(cw == m, J, _SENT), axis=(1, 2), keepdims=True)
        cw = jnp.where((cw == m) & (J == jm), _BIG, cw)
        acc_i = jnp.where(lane_k == k, jm.reshape(rows, 1), acc_i)
        acc_v = jnp.where(lane_k == k, m.reshape(rows, 1), acc_v)
        return cw, acc_i, acc_v

    acc_i0 = jnp.zeros((rows, KNN), jnp.int32)
    acc_v0 = jnp.zeros((rows, KNN), jnp.float32)
    _, acc_i, acc_v = lax.fori_loop(0, KNN, bbody, (C, acc_i0, acc_v0))

    t16 = acc_v[:, KNN - 1:KNN].reshape(rows, 1, 1)
    j16 = acc_i[:, KNN - 1:KNN].reshape(rows, 1, 1)
    less = (d2 < t16) | ((d2 == t16) & (col3 < j16))
    cnt = jnp.sum(less.astype(jnp.int32), axis=(1, 2))
    ok = jnp.all(cnt == 15)

    idx_ref[0] = acc_i + b * n

    @pl.when(jnp.logical_not(ok))
    def _fallback():
        def fbody(k, carry):
            d2c, acc = carry
            m = jnp.min(d2c, axis=(1, 2), keepdims=True)
            j = jnp.min(jnp.where(d2c == m, col3, _SENT), axis=(1, 2),
                        keepdims=True)
            d2c = jnp.where(col3 == j, _BIG, d2c)
            acc = jnp.where(lane_k == k, j.reshape(rows, 1), acc)
            return d2c, acc

        _, acc = lax.fori_loop(0, KNN, fbody, (d2, acc_i0))
        idx_ref[0] = acc + b * n


def _topk_indices(xsr, ysr, xsc, ysc, *, b, n, rows):
    grid = (b, n // rows)
    return pl.pallas_call(
        functools.partial(_topk_body, n=n, rows=rows),
        grid=grid,
        in_specs=[
            pl.BlockSpec((1, rows, 1), lambda bb, i: (bb, i, 0)),
            pl.BlockSpec((1, rows, 1), lambda bb, i: (bb, i, 0)),
            pl.BlockSpec((1, 1, n), lambda bb, i: (bb, 0, 0)),
            pl.BlockSpec((1, 1, n), lambda bb, i: (bb, 0, 0)),
        ],
        out_specs=pl.BlockSpec((1, rows, KNN), lambda bb, i: (bb, i, 0)),
        out_shape=jax.ShapeDtypeStruct((b, n, KNN), jnp.int32),
    )(xsr, ysr, xsc, ysc)


def _sc_gather(table, flat_idx, *, g, d, chunk=128):
    g_per_w = g // _NW
    mesh = plsc.VectorSubcoreMesh(core_axis_name="c", subcore_axis_name="s")

    @functools.partial(
        pl.kernel,
        mesh=mesh,
        out_type=jax.ShapeDtypeStruct((g, d), jnp.float32),
        scratch_types=[
            pltpu.VMEM((chunk,), jnp.int32),
            pltpu.VMEM((chunk, d), jnp.float32),
            pltpu.SemaphoreType.DMA,
        ],
    )
    def gather_kernel(tab_hbm, idx_hbm, out_hbm, idx_v, rows_v, sem):
        wid = lax.axis_index("s") * _NC + lax.axis_index("c")

        def body(i, carry):
            base = wid * g_per_w + i * chunk
            pltpu.sync_copy(idx_hbm.at[pl.ds(base, chunk)], idx_v)
            pltpu.async_copy(tab_hbm.at[idx_v], rows_v, sem).wait()
            pltpu.sync_copy(rows_v, out_hbm.at[pl.ds(base, chunk)])
            return carry

        lax.fori_loop(0, g_per_w // chunk, body, 0)

    return gather_kernel(table, flat_idx)


def _mlp_body(g_ref, own_ref, w_ref, out_ref):
    xr = own_ref[:, 0:1]
    yr = own_ref[:, 1:2]
    px = g_ref[:, 64:65]
    py = g_ref[:, 65:66]
    f = g_ref[:, 0:64]
    dx = xr - px
    dy = yr - py
    nrm = jnp.sqrt(dx * dx + dy * dy + 1e-12)
    wa = w_ref[0:1, :]
    wb = w_ref[1:2, :]
    wc = w_ref[2:3, :]
    wd = w_ref[3:4, :]
    we = w_ref[4:5, :]
    bias = w_ref[5:6, :]
    r = xr * wa + yr * wb + px * wc + py * wd + nrm * we + bias
    r = jnp.maximum(r, 0.0)
    out_ref[...] = jnp.concatenate([f, r], axis=1)


def _mlp(g2, own_exp, wpack, *, m, rows):
    grid = (m // rows,)
    return pl.pallas_call(
        _mlp_body,
        grid=grid,
        in_specs=[
            pl.BlockSpec((rows, 128), lambda i: (i, 0)),
            pl.BlockSpec((rows, 2), lambda i: (i, 0)),
            pl.BlockSpec((8, 64), lambda i: (0, 0)),
        ],
        out_specs=pl.BlockSpec((rows, 128), lambda i: (i, 0)),
        out_shape=jax.ShapeDtypeStruct((m, 128), jnp.float32),
    )(g2, own_exp, wpack)


def kernel(pc, feats, W, b):
    B, N, dims = pc.shape
    f_dims = feats.shape[-1]
    G = B * N * KNN

    xs = pc[..., 0]
    ys = pc[..., 1]
    xsr = xs[..., None]
    ysr = ys[..., None]
    xsc = xs[:, None, :]
    ysc = ys[:, None, :]

    idx = _topk_indices(xsr, ysr, xsc, ysc, b=B, n=N, rows=128)
    flat_idx = idx.reshape(G)

    pc2 = pc.reshape(B * N, dims)
    table = jnp.concatenate(
        [feats.reshape(B * N, f_dims), pc2,
         jnp.zeros((B * N, 128 - f_dims - dims), jnp.float32)],
        axis=1,
    )

    g = _sc_gather(table, flat_idx, g=G, d=128)

    wpack = jnp.stack(
        [W[0] + W[4], W[1] + W[5], W[2] - W[4], W[3] - W[5], W[6], b,
         jnp.zeros_like(b), jnp.zeros_like(b)],
        axis=0,
    )

    own_exp = jnp.broadcast_to(pc2[:, None, :], (B * N, KNN, dims)).reshape(G, dims)

    out = _mlp(g, own_exp, wpack, m=G, rows=2048)
    return out.reshape(B, N, KNN, 2 * f_dims)


# speedup vs baseline: 2.2660x; 2.2660x over previous
import functools

import jax
import jax.numpy as jnp
from jax import lax
from jax.experimental import pallas as pl
from jax.experimental.pallas import tpu as pltpu
from jax.experimental.pallas import tpu_sc as plsc

KNN = 16
_BIG = 3.0e38

_NC, _NS = 2, 16
_NW = _NC * _NS


def _rn_bf16(v):
    u = lax.bitcast_convert_type(v, jnp.int32)
    odd = lax.shift_right_logical(u, 16) & 1
    r = (u + 32767 + odd) & jnp.int32(-65536)
    return lax.bitcast_convert_type(r, jnp.float32)


def _topk_body(xsr_ref, ysr_ref, xsc_ref, ysc_ref, idx_ref, d2_ref, *, n, rows):
    b = pl.program_id(0)
    grps = n // 128
    xr = xsr_ref[0].reshape(rows, 1, 1)
    yr = ysr_ref[0].reshape(rows, 1, 1)
    xc = xsc_ref[0].reshape(1, grps, 128)
    yc = ysc_ref[0].reshape(1, grps, 128)
    sqr = xr * xr + yr * yr
    sqc = xc * xc + yc * yc
    xrb, yrb, xcb, ycb = _rn_bf16(xr), _rn_bf16(yr), _rn_bf16(xc), _rn_bf16(yc)
    d2_ref[...] = sqr + sqc - 2.0 * (xrb * xcb + yrb * ycb)

    a_iota = lax.broadcasted_iota(jnp.int32, (rows, grps, 128), 1)
    l_iota = lax.broadcasted_iota(jnp.int32, (rows, grps, 128), 2)
    col3 = a_iota * 128 + l_iota
    lane_k = lax.broadcasted_iota(jnp.int32, (rows, KNN), 1)
    _SENT = jnp.int32(1 << 30)

    cand_v, cand_j = [], []
    gm = jnp.full((rows, 1, 128), -_BIG, jnp.float32)
    for _ in range(4):
        dm = jnp.where(d2_ref[...] > gm, d2_ref[...], _BIG)
        m = jnp.min(dm, axis=1, keepdims=True)
        asel = jnp.where(dm == m, a_iota, grps)
        a_star = jnp.min(asel, axis=1, keepdims=True)
        cand_v.append(m)
        cand_j.append(a_star * 128 + l_iota[:, :1, :])
        gm = m
    C = jnp.concatenate(cand_v, axis=1)
    J = jnp.concatenate(cand_j, axis=1)

    def bbody(k, carry):
        cw, acc_i, acc_v = carry
        m = jnp.min(cw, axis=(1, 2), keepdims=True)
        jm = jnp.min(jnp.where(cw == m, J, _SENT), axis=(1, 2), keepdims=True)
        cw = jnp.where((cw == m) & (J == jm), _BIG, cw)
        acc_i = jnp.where(lane_k == k, jm.reshape(rows, 1), acc_i)
        acc_v = jnp.where(lane_k == k, m.reshape(rows, 1), acc_v)
        return cw, acc_i, acc_v

    acc_i0 = jnp.zeros((rows, KNN), jnp.int32)
    acc_v0 = jnp.zeros((rows, KNN), jnp.float32)
    _, acc_i, acc_v = lax.fori_loop(0, KNN, bbody, (C, acc_i0, acc_v0))

    t16 = acc_v[:, KNN - 1:KNN].reshape(rows, 1, 1)
    j16 = acc_i[:, KNN - 1:KNN].reshape(rows, 1, 1)
    d2v = d2_ref[...]
    less = (d2v < t16) | ((d2v == t16) & (col3 < j16))
    cnt = jnp.sum(less.astype(jnp.int32), axis=(1, 2))
    ok = jnp.all(cnt == 15)

    idx_ref[0] = acc_i + b * n

    @pl.when(jnp.logical_not(ok))
    def _fallback():
        def fbody(k, acc):
            d2c = d2_ref[...]
            m = jnp.min(d2c, axis=(1, 2), keepdims=True)
            j = jnp.min(jnp.where(d2c == m, col3, _SENT), axis=(1, 2),
                        keepdims=True)
            d2_ref[...] = jnp.where(col3 == j, _BIG, d2c)
            return jnp.where(lane_k == k, j.reshape(rows, 1), acc)

        acc = lax.fori_loop(0, KNN, fbody, acc_i0)
        idx_ref[0] = acc + b * n


def _topk_indices(xsr, ysr, xsc, ysc, *, b, n, rows):
    grid = (b, n // rows)
    return pl.pallas_call(
        functools.partial(_topk_body, n=n, rows=rows),
        grid=grid,
        in_specs=[
            pl.BlockSpec((1, rows, 1), lambda bb, i: (bb, i, 0)),
            pl.BlockSpec((1, rows, 1), lambda bb, i: (bb, i, 0)),
            pl.BlockSpec((1, 1, n), lambda bb, i: (bb, 0, 0)),
            pl.BlockSpec((1, 1, n), lambda bb, i: (bb, 0, 0)),
        ],
        out_specs=pl.BlockSpec((1, rows, KNN), lambda bb, i: (bb, i, 0)),
        out_shape=jax.ShapeDtypeStruct((b, n, KNN), jnp.int32),
        scratch_shapes=[pltpu.VMEM((rows, n // 128, 128), jnp.float32)],
    )(xsr, ysr, xsc, ysc)


def _sc_gather(table, flat_idx, *, g, d, chunk=128):
    g_per_w = g // _NW
    mesh = plsc.VectorSubcoreMesh(core_axis_name="c", subcore_axis_name="s")

    @functools.partial(
        pl.kernel,
        mesh=mesh,
        out_type=jax.ShapeDtypeStruct((g, d), jnp.float32),
        scratch_types=[
            pltpu.VMEM((chunk,), jnp.int32),
            pltpu.VMEM((chunk, d), jnp.float32),
            pltpu.SemaphoreType.DMA,
        ],
    )
    def gather_kernel(tab_hbm, idx_hbm, out_hbm, idx_v, rows_v, sem):
        wid = lax.axis_index("s") * _NC + lax.axis_index("c")

        def body(i, carry):
            base = wid * g_per_w + i * chunk
            pltpu.sync_copy(idx_hbm.at[pl.ds(base, chunk)], idx_v)
            pltpu.async_copy(tab_hbm.at[idx_v], rows_v, sem).wait()
            pltpu.sync_copy(rows_v, out_hbm.at[pl.ds(base, chunk)])
            return carry

        lax.fori_loop(0, g_per_w // chunk, body, 0)

    return gather_kernel(table, flat_idx)


def _mlp_body(g_ref, own_ref, w_ref, out_ref):
    xr = own_ref[:, 0:1]
    yr = own_ref[:, 1:2]
    px = g_ref[:, 64:65]
    py = g_ref[:, 65:66]
    f = g_ref[:, 0:64]
    dx = xr - px
    dy = yr - py
    nrm = jnp.sqrt(dx * dx + dy * dy + 1e-12)
    wa = w_ref[0:1, :]
    wb = w_ref[1:2, :]
    wc = w_ref[2:3, :]
    wd = w_ref[3:4, :]
    we = w_ref[4:5, :]
    bias = w_ref[5:6, :]
    r = xr * wa + yr * wb + px * wc + py * wd + nrm * we + bias
    r = jnp.maximum(r, 0.0)
    out_ref[...] = jnp.concatenate([f, r], axis=1)


def _mlp(g2, own_exp, wpack, *, m, rows):
    grid = (m // rows,)
    return pl.pallas_call(
        _mlp_body,
        grid=grid,
        in_specs=[
            pl.BlockSpec((rows, 128), lambda i: (i, 0)),
            pl.BlockSpec((rows, 2), lambda i: (i, 0)),
            pl.BlockSpec((8, 64), lambda i: (0, 0)),
        ],
        out_specs=pl.BlockSpec((rows, 128), lambda i: (i, 0)),
        out_shape=jax.ShapeDtypeStruct((m, 128), jnp.float32),
    )(g2, own_exp, wpack)


def kernel(pc, feats, W, b):
    B, N, dims = pc.shape
    f_dims = feats.shape[-1]
    G = B * N * KNN

    xs = pc[..., 0]
    ys = pc[..., 1]
    xsr = xs[..., None]
    ysr = ys[..., None]
    xsc = xs[:, None, :]
    ysc = ys[:, None, :]

    idx = _topk_indices(xsr, ysr, xsc, ysc, b=B, n=N, rows=256)
    flat_idx = idx.reshape(G)

    pc2 = pc.reshape(B * N, dims)
    table = jnp.concatenate(
        [feats.reshape(B * N, f_dims), pc2,
         jnp.zeros((B * N, 128 - f_dims - dims), jnp.float32)],
        axis=1,
    )

    g = _sc_gather(table, flat_idx, g=G, d=128)

    wpack = jnp.stack(
        [W[0] + W[4], W[1] + W[5], W[2] - W[4], W[3] - W[5], W[6], b,
         jnp.zeros_like(b), jnp.zeros_like(b)],
        axis=0,
    )

    own_exp = jnp.broadcast_to(pc2[:, None, :], (B * N, KNN, dims)).reshape(G, dims)

    out = _mlp(g, own_exp, wpack, m=G, rows=2048)
    return out.reshape(B, N, KNN, 2 * f_dims)


# speedup vs baseline: 2.8858x; 1.2735x over previous
import functools

import jax
import jax.numpy as jnp
from jax import lax
from jax.experimental import pallas as pl
from jax.experimental.pallas import tpu as pltpu
from jax.experimental.pallas import tpu_sc as plsc

KNN = 16
_BIG = 3.0e38

_NC, _NS = 2, 16
_NW = _NC * _NS


def _rn_bf16(v):
    u = lax.bitcast_convert_type(v, jnp.int32)
    odd = lax.shift_right_logical(u, 16) & 1
    r = (u + 32767 + odd) & jnp.int32(-65536)
    return lax.bitcast_convert_type(r, jnp.float32)


def _topk_body(xsr_ref, ysr_ref, xsc_ref, ysc_ref, idx_ref, *, n, rows):
    b = pl.program_id(0)
    xr = xsr_ref[0]
    yr = ysr_ref[0]
    xc = xsc_ref[0]
    yc = ysc_ref[0]
    sqr = xr * xr + yr * yr
    sqc = xc * xc + yc * yc
    xrb, yrb, xcb, ycb = _rn_bf16(xr), _rn_bf16(yr), _rn_bf16(xc), _rn_bf16(yc)
    d2 = sqr + sqc - 2.0 * (xrb * xcb + yrb * ycb)

    colidx = lax.broadcasted_iota(jnp.int32, (rows, n), 1)
    lane_k = lax.broadcasted_iota(jnp.int32, (1, KNN), 1)

    def body(k, carry):
        d2c, acc = carry
        m = jnp.min(d2c, axis=1, keepdims=True)
        cand = jnp.where(d2c == m, colidx, n)
        j = jnp.min(cand, axis=1, keepdims=True)
        d2c = jnp.where(colidx == j, _BIG, d2c)
        acc = jnp.where(lane_k == k, j, acc)
        return d2c, acc

    acc0 = jnp.zeros((rows, KNN), jnp.int32)
    _, acc = lax.fori_loop(0, KNN, body, (d2, acc0))
    idx_ref[0] = acc + b * n


def _topk_indices(xsr, ysr, xsc, ysc, *, b, n, rows):
    grid = (b, n // rows)
    return pl.pallas_call(
        functools.partial(_topk_body, n=n, rows=rows),
        grid=grid,
        in_specs=[
            pl.BlockSpec((1, rows, 1), lambda bb, i: (bb, i, 0)),
            pl.BlockSpec((1, rows, 1), lambda bb, i: (bb, i, 0)),
            pl.BlockSpec((1, 1, n), lambda bb, i: (bb, 0, 0)),
            pl.BlockSpec((1, 1, n), lambda bb, i: (bb, 0, 0)),
        ],
        out_specs=pl.BlockSpec((1, rows, KNN), lambda bb, i: (bb, i, 0)),
        out_shape=jax.ShapeDtypeStruct((b, n, KNN), jnp.int32),
    )(xsr, ysr, xsc, ysc)


def _sc_gather(table, flat_idx, *, g, d, chunk=128):
    g_per_w = g // _NW
    mesh = plsc.VectorSubcoreMesh(core_axis_name="c", subcore_axis_name="s")

    @functools.partial(
        pl.kernel,
        mesh=mesh,
        out_type=jax.ShapeDtypeStruct((g, d), jnp.float32),
        scratch_types=[
            pltpu.VMEM((chunk,), jnp.int32),
            pltpu.VMEM((chunk, d), jnp.float32),
            pltpu.SemaphoreType.DMA,
        ],
    )
    def gather_kernel(tab_hbm, idx_hbm, out_hbm, idx_v, rows_v, sem):
        wid = lax.axis_index("s") * _NC + lax.axis_index("c")

        def body(i, carry):
            base = wid * g_per_w + i * chunk
            pltpu.sync_copy(idx_hbm.at[pl.ds(base, chunk)], idx_v)
            pltpu.async_copy(tab_hbm.at[idx_v], rows_v, sem).wait()
            pltpu.sync_copy(rows_v, out_hbm.at[pl.ds(base, chunk)])
            return carry

        lax.fori_loop(0, g_per_w // chunk, body, 0)

    return gather_kernel(table, flat_idx)


def _mlp_body(g_ref, own_ref, w_ref, out_ref):
    xr = own_ref[:, 0:1]
    yr = own_ref[:, 1:2]
    px = g_ref[:, 64:65]
    py = g_ref[:, 65:66]
    f = g_ref[:, 0:64]
    dx = xr - px
    dy = yr - py
    nrm = jnp.sqrt(dx * dx + dy * dy + 1e-12)
    wa = w_ref[0:1, :]
    wb = w_ref[1:2, :]
    wc = w_ref[2:3, :]
    wd = w_ref[3:4, :]
    we = w_ref[4:5, :]
    bias = w_ref[5:6, :]
    r = xr * wa + yr * wb + px * wc + py * wd + nrm * we + bias
    r = jnp.maximum(r, 0.0)
    out_ref[...] = jnp.concatenate([f, r], axis=1)


def _mlp(g2, own_exp, wpack, *, m, rows):
    grid = (m // rows,)
    return pl.pallas_call(
        _mlp_body,
        grid=grid,
        in_specs=[
            pl.BlockSpec((rows, 128), lambda i: (i, 0)),
            pl.BlockSpec((rows, 2), lambda i: (i, 0)),
            pl.BlockSpec((8, 64), lambda i: (0, 0)),
        ],
        out_specs=pl.BlockSpec((rows, 128), lambda i: (i, 0)),
        out_shape=jax.ShapeDtypeStruct((m, 128), jnp.float32),
    )(g2, own_exp, wpack)


def kernel(pc, feats, W, b):
    B, N, dims = pc.shape
    f_dims = feats.shape[-1]
    G = B * N * KNN

    xs = pc[..., 0]
    ys = pc[..., 1]
    xsr = xs[..., None]
    ysr = ys[..., None]
    xsc = xs[:, None, :]
    ysc = ys[:, None, :]

    idx = _topk_indices(xsr, ysr, xsc, ysc, b=B, n=N, rows=256)
    flat_idx = idx.reshape(G)

    pc2 = pc.reshape(B * N, dims)
    table = jnp.concatenate(
        [feats.reshape(B * N, f_dims), pc2,
         jnp.zeros((B * N, 128 - f_dims - dims), jnp.float32)],
        axis=1,
    )

    g = _sc_gather(table, flat_idx, g=G, d=128)

    wpack = jnp.stack(
        [W[0] + W[4], W[1] + W[5], W[2] - W[4], W[3] - W[5], W[6], b,
         jnp.zeros_like(b), jnp.zeros_like(b)],
        axis=0,
    )

    own_exp = jnp.broadcast_to(pc2[:, None, :], (B * N, KNN, dims)).reshape(G, dims)

    out = _mlp(g, own_exp, wpack, m=G, rows=2048)
    return out.reshape(B, N, KNN, 2 * f_dims)


# speedup vs baseline: 3.0033x; 1.0407x over previous
import functools

import jax
import jax.numpy as jnp
from jax import lax
from jax.experimental import pallas as pl
from jax.experimental.pallas import tpu as pltpu
from jax.experimental.pallas import tpu_sc as plsc

KNN = 16
_BIG = 3.0e38

_NC, _NS = 2, 16
_NW = _NC * _NS


def _rn_bf16(v):
    u = lax.bitcast_convert_type(v, jnp.int32)
    odd = lax.shift_right_logical(u, 16) & 1
    r = (u + 32767 + odd) & jnp.int32(-65536)
    return lax.bitcast_convert_type(r, jnp.float32)


def _topk_body(xsr_ref, ysr_ref, xsc_ref, ysc_ref, idx_ref, *, n, rows):
    b = pl.program_id(0)
    xr = xsr_ref[0]
    yr = ysr_ref[0]
    xc = xsc_ref[0]
    yc = ysc_ref[0]
    sqr = xr * xr + yr * yr
    sqc = xc * xc + yc * yc
    xrb, yrb, xcb, ycb = _rn_bf16(xr), _rn_bf16(yr), _rn_bf16(xc), _rn_bf16(yc)
    d2 = sqr + sqc - 2.0 * (xrb * xcb + yrb * ycb)

    colidx = lax.broadcasted_iota(jnp.int32, (rows, n), 1)
    lane_k = lax.broadcasted_iota(jnp.int32, (1, KNN), 1)

    def body(k, carry):
        d2c, acc = carry
        m = jnp.min(d2c, axis=1, keepdims=True)
        cand = jnp.where(d2c == m, colidx, n)
        j = jnp.min(cand, axis=1, keepdims=True)
        d2c = jnp.where(colidx == j, _BIG, d2c)
        acc = jnp.where(lane_k == k, j, acc)
        return d2c, acc

    acc0 = jnp.zeros((rows, KNN), jnp.int32)
    _, acc = lax.fori_loop(0, KNN, body, (d2, acc0))
    idx_ref[0] = acc + b * n


def _topk_indices(xsr, ysr, xsc, ysc, *, b, n, rows):
    grid = (b, n // rows)
    return pl.pallas_call(
        functools.partial(_topk_body, n=n, rows=rows),
        grid=grid,
        in_specs=[
            pl.BlockSpec((1, rows, 1), lambda bb, i: (bb, i, 0)),
            pl.BlockSpec((1, rows, 1), lambda bb, i: (bb, i, 0)),
            pl.BlockSpec((1, 1, n), lambda bb, i: (bb, 0, 0)),
            pl.BlockSpec((1, 1, n), lambda bb, i: (bb, 0, 0)),
        ],
        out_specs=pl.BlockSpec((1, rows, KNN), lambda bb, i: (bb, i, 0)),
        out_shape=jax.ShapeDtypeStruct((b, n, KNN), jnp.int32),
    )(xsr, ysr, xsc, ysc)


def _sc_gather(table, flat_idx, *, g, d, chunk=128):
    g_per_w = g // _NW
    mesh = plsc.VectorSubcoreMesh(core_axis_name="c", subcore_axis_name="s")

    @functools.partial(
        pl.kernel,
        mesh=mesh,
        out_type=jax.ShapeDtypeStruct((g, d), jnp.float32),
        scratch_types=[
            pltpu.VMEM((chunk,), jnp.int32),
            pltpu.VMEM((chunk, d), jnp.float32),
            pltpu.SemaphoreType.DMA,
        ],
    )
    def gather_kernel(tab_hbm, idx_hbm, out_hbm, idx_v, rows_v, sem):
        wid = lax.axis_index("s") * _NC + lax.axis_index("c")

        def body(i, carry):
            base = wid * g_per_w + i * chunk
            pltpu.sync_copy(idx_hbm.at[pl.ds(base, chunk)], idx_v)
            pltpu.async_copy(tab_hbm.at[idx_v], rows_v, sem).wait()
            pltpu.sync_copy(rows_v, out_hbm.at[pl.ds(base, chunk)])
            return carry

        lax.fori_loop(0, g_per_w // chunk, body, 0)

    return gather_kernel(table, flat_idx)


def _mlp_body(g_ref, own_ref, w_ref, out_ref):
    xr = own_ref[:, 0:1]
    yr = own_ref[:, 1:2]
    px = g_ref[:, 64:65]
    py = g_ref[:, 65:66]
    f = g_ref[:, 0:64]
    dx = xr - px
    dy = yr - py
    nrm = jnp.sqrt(dx * dx + dy * dy + 1e-12)
    wa = w_ref[0:1, :]
    wb = w_ref[1:2, :]
    wc = w_ref[2:3, :]
    wd = w_ref[3:4, :]
    we = w_ref[4:5, :]
    bias = w_ref[5:6, :]
    r = xr * wa + yr * wb + px * wc + py * wd + nrm * we + bias
    r = jnp.maximum(r, 0.0)
    out_ref[...] = jnp.concatenate([f, r], axis=1)


def _mlp(g2, own_exp, wpack, *, m, rows):
    grid = (m // rows,)
    return pl.pallas_call(
        _mlp_body,
        grid=grid,
        in_specs=[
            pl.BlockSpec((rows, 128), lambda i: (i, 0)),
            pl.BlockSpec((rows, 2), lambda i: (i, 0)),
            pl.BlockSpec((8, 64), lambda i: (0, 0)),
        ],
        out_specs=pl.BlockSpec((rows, 128), lambda i: (i, 0)),
        out_shape=jax.ShapeDtypeStruct((m, 128), jnp.float32),
    )(g2, own_exp, wpack)


def kernel(pc, feats, W, b):
    B, N, dims = pc.shape
    f_dims = feats.shape[-1]
    G = B * N * KNN

    xs = pc[..., 0]
    ys = pc[..., 1]
    xsr = xs[..., None]
    ysr = ys[..., None]
    xsc = xs[:, None, :]
    ysc = ys[:, None, :]

    idx = _topk_indices(xsr, ysr, xsc, ysc, b=B, n=N, rows=512)
    flat_idx = idx.reshape(G)

    pc2 = pc.reshape(B * N, dims)
    table = jnp.concatenate(
        [feats.reshape(B * N, f_dims), pc2,
         jnp.zeros((B * N, 128 - f_dims - dims), jnp.float32)],
        axis=1,
    )

    g = _sc_gather(table, flat_idx, g=G, d=128)

    wpack = jnp.stack(
        [W[0] + W[4], W[1] + W[5], W[2] - W[4], W[3] - W[5], W[6], b,
         jnp.zeros_like(b), jnp.zeros_like(b)],
        axis=0,
    )

    own_exp = jnp.broadcast_to(pc2[:, None, :], (B * N, KNN, dims)).reshape(G, dims)

    out = _mlp(g, own_exp, wpack, m=G, rows=2048)
    return out.reshape(B, N, KNN, 2 * f_dims)
